# Initial kernel scaffold; baseline (speedup 1.0000x reference)
import jax
import jax.numpy as jnp
from jax.experimental import pallas as pl


def kernel(indices, word_embeddings):
    raise NotImplementedError("write your pallas kernel here")


# speedup vs baseline: 3.7053x; 3.7053x over previous
import functools

import jax
import jax.numpy as jnp
from jax import lax
from jax.experimental import pallas as pl
from jax.experimental.pallas import tpu as pltpu
from jax.experimental.pallas import tpu_sc as plsc

_VOCAB = 100
_HIDDEN = 128
_LANES = 128

_NC = 2
_NS = 16
_NW = _NC * _NS

_K = 4


def _gather_body(idx_hbm, tbl_hbm, out_hbm, idx_v, rows_v, sem):
    n_rows = idx_hbm.shape[0]
    per_w = n_rows // _NW
    chunks = per_w // _K
    wid = lax.axis_index("s") * _NC + lax.axis_index("c")
    base = wid * per_w

    def step(j, carry):
        jb = base + j * _K
        pltpu.sync_copy(idx_hbm.at[pl.ds(jb, _K)], idx_v)
        copies = [
            pltpu.async_copy(tbl_hbm.at[idx_v.at[k]], rows_v.at[k], sem)
            for k in range(_K)
        ]
        for c in copies:
            c.wait()
        pltpu.sync_copy(rows_v, out_hbm.at[pl.ds(jb, _K)])
        return carry

    lax.fori_loop(0, chunks, step, 0)


@jax.jit
def _lookup(idx2d, table):
    n_rows = idx2d.shape[0]
    mesh = plsc.VectorSubcoreMesh(core_axis_name="c", subcore_axis_name="s")
    return pl.kernel(
        _gather_body,
        mesh=mesh,
        out_type=jax.ShapeDtypeStruct((n_rows, _LANES, _HIDDEN), jnp.float32),
        scratch_types=[
            pltpu.VMEM((_K, _LANES), jnp.int32),
            pltpu.VMEM((_K, _LANES, _HIDDEN), jnp.float32),
            pltpu.SemaphoreType.DMA,
        ],
    )(idx2d, table)


def kernel(indices, word_embeddings):
    b, t = indices.shape
    flat = b * t
    idx2d = indices.reshape(flat // _LANES, _LANES).astype(jnp.int32)
    out = _lookup(idx2d, word_embeddings)
    return out.reshape(b, t, _HIDDEN)
